# Initial kernel scaffold; baseline (speedup 1.0000x reference)
import functools

import jax
import jax.numpy as jnp
from jax import lax
from jax.experimental import pallas as pl
from jax.experimental.pallas import tpu as pltpu
from jax.experimental.pallas import tpu_sc as plsc

B = 16384
L = 20
V = 1000000
D = 32
BL = B * L

NC = 2
NS = 16
NW = NC * NS

CH = 512

F32 = jnp.float32


def _sc_gather(Eum, Eim, Eumf, Eimf, uid, posf, negf):
    mesh = plsc.VectorSubcoreMesh(core_axis_name="c", subcore_axis_name="s")
    out_type = (
        jax.ShapeDtypeStruct((B, D), F32),
        jax.ShapeDtypeStruct((B, D), F32),
        jax.ShapeDtypeStruct((BL, D), F32),
        jax.ShapeDtypeStruct((BL, D), F32),
        jax.ShapeDtypeStruct((BL, D), F32),
        jax.ShapeDtypeStruct((BL, D), F32),
    )

    @functools.partial(
        pl.kernel,
        mesh=mesh,
        out_type=out_type,
        scratch_types=[
            pltpu.VMEM((CH,), jnp.int32),
            pltpu.VMEM((CH, D), F32),
            pltpu.VMEM((CH, D), F32),
            pltpu.SemaphoreType.DMA,
            pltpu.SemaphoreType.DMA,
        ],
    )
    def k(eum, eim, eumf, eimf, uid_h, pos_h, neg_h,
          o_ue_mlp, o_ue_mf, o_pos_mlp, o_neg_mlp, o_pos_mf, o_neg_mf,
          idx_v, r0, r1, sem0, sem1):
        wid = lax.axis_index("s") * NC + lax.axis_index("c")
        jobs = (
            (uid_h, B // NW, eum, o_ue_mlp, eumf, o_ue_mf),
            (pos_h, BL // NW, eim, o_pos_mlp, eimf, o_pos_mf),
            (neg_h, BL // NW, eim, o_neg_mlp, eimf, o_neg_mf),
        )
        for idx_h, n_per_w, t0, o0, t1, o1 in jobs:
            nchunks = n_per_w // CH

            def body(c, _, idx_h=idx_h, n_per_w=n_per_w,
                     t0=t0, o0=o0, t1=t1, o1=o1):
                base = wid * n_per_w + c * CH
                pltpu.sync_copy(idx_h.at[pl.ds(base, CH)], idx_v)
                cp0 = pltpu.async_copy(t0.at[idx_v], r0, sem0)
                cp1 = pltpu.async_copy(t1.at[idx_v], r1, sem1)
                cp0.wait()
                cp1.wait()
                pltpu.sync_copy(r0, o0.at[pl.ds(base, CH)])
                pltpu.sync_copy(r1, o1.at[pl.ds(base, CH)])
                return 0

            lax.fori_loop(0, nchunks, body, 0)

    return k(Eum, Eim, Eumf, Eimf, uid, posf, negf)


BB = 256
RR = 5 * BB


def _tc_body(ue_mlp, ue_mf, pos_mlp, neg_mlp, pos_mf, neg_mf,
             w0ut, a0, a1, a2, aom, aof, b0t, b1t, b2t, bot,
             pos_out, neg_out):
    hu = jnp.dot(ue_mlp[:], w0ut[:], preferred_element_type=F32)
    hu4 = jnp.concatenate([hu, hu, hu, hu], axis=1)
    hu_rep = jnp.broadcast_to(hu4[:, None, :], (BB, 5, 4 * D)).reshape(RR, 4 * D)
    uf = ue_mf[:]
    uf4 = jnp.concatenate([uf, uf, uf, uf], axis=1)
    uf_rep = jnp.broadcast_to(uf4[:, None, :], (BB, 5, 4 * D)).reshape(RR, 4 * D)

    def tower(ie_mlp, ie_mf):
        h = jnp.maximum(
            jnp.dot(ie_mlp, a0[:], preferred_element_type=F32) + hu_rep + b0t[:], 0.0)
        h = jnp.maximum(
            jnp.dot(h, a1[:], preferred_element_type=F32) + b1t[:], 0.0)
        h = jnp.maximum(
            jnp.dot(h, a2[:], preferred_element_type=F32) + b2t[:], 0.0)
        return (jnp.dot(h, aom[:], preferred_element_type=F32)
                + jnp.dot(uf_rep * ie_mf, aof[:], preferred_element_type=F32)
                + bot[:])

    pos_out[:] = tower(pos_mlp[:], pos_mf[:])
    neg_out[:] = tower(neg_mlp[:], neg_mf[:])


def kernel(Eum, Eim, Eumf, Eimf, W0, b0, W1, b1, W2, b2, Wo, bo, uid, pos, neg):
    posf = pos.reshape(-1)
    negf = neg.reshape(-1)
    ue_mlp, ue_mf, pos_mlp, neg_mlp, pos_mf, neg_mf = _sc_gather(
        Eum, Eim, Eumf, Eimf, uid, posf, negf)

    pos_mlp_p = pos_mlp.reshape(BL // 4, 4 * D)
    neg_mlp_p = neg_mlp.reshape(BL // 4, 4 * D)
    pos_mf_p = pos_mf.reshape(BL // 4, 4 * D)
    neg_mf_p = neg_mf.reshape(BL // 4, 4 * D)

    eye4 = jnp.eye(4, dtype=F32)
    w0ut = W0[:, :D].T
    a0 = jnp.kron(eye4, W0[:, D:].T)
    a1 = jnp.kron(eye4, W1.T)
    a2 = jnp.kron(eye4, W2.T)
    aom = jnp.kron(eye4, Wo[:, :D].T)
    aof = jnp.kron(eye4, Wo[:, D:].T)
    b0t = jnp.tile(b0, 4)[None]
    b1t = jnp.tile(b1, 4)[None]
    b2t = jnp.tile(b2, 4)[None]
    bot = jnp.tile(bo, 4)[None]

    grid = B // BB
    full = lambda shape: pl.BlockSpec(shape, lambda i: (0, 0))
    pos_out, neg_out = pl.pallas_call(
        _tc_body,
        grid=(grid,),
        in_specs=[
            pl.BlockSpec((BB, D), lambda i: (i, 0)),
            pl.BlockSpec((BB, D), lambda i: (i, 0)),
            pl.BlockSpec((RR, 4 * D), lambda i: (i, 0)),
            pl.BlockSpec((RR, 4 * D), lambda i: (i, 0)),
            pl.BlockSpec((RR, 4 * D), lambda i: (i, 0)),
            pl.BlockSpec((RR, 4 * D), lambda i: (i, 0)),
            full((D, D)), full((4 * D, 4 * D)), full((4 * D, 4 * D)),
            full((4 * D, 4 * D)), full((4 * D, 4)), full((4 * D, 4)),
            full((1, 4 * D)), full((1, 4 * D)), full((1, 4 * D)), full((1, 4)),
        ],
        out_specs=[
            pl.BlockSpec((RR, 4), lambda i: (i, 0)),
            pl.BlockSpec((RR, 4), lambda i: (i, 0)),
        ],
        out_shape=[
            jax.ShapeDtypeStruct((BL // 4, 4), F32),
            jax.ShapeDtypeStruct((BL // 4, 4), F32),
        ],
        compiler_params=pltpu.CompilerParams(
            dimension_semantics=("arbitrary",),
        ),
    )(ue_mlp, ue_mf, pos_mlp_p, neg_mlp_p, pos_mf_p, neg_mf_p,
      w0ut, a0, a1, a2, aom, aof, b0t, b1t, b2t, bot)

    return (pos_out.reshape(B, L, 1), neg_out.reshape(B, L, 1))


# speedup vs baseline: 15.8183x; 15.8183x over previous
import functools

import jax
import jax.numpy as jnp
from jax import lax
from jax.experimental import pallas as pl
from jax.experimental.pallas import tpu as pltpu
from jax.experimental.pallas import tpu_sc as plsc

B = 16384
L = 20
V = 1000000
D = 32
BL = B * L

NC = 2
NS = 16
NW = NC * NS

CH = 512

F32 = jnp.float32


def _sc_gather(Eum, Eim, Eumf, Eimf, uid, posf, negf):
    mesh = plsc.VectorSubcoreMesh(core_axis_name="c", subcore_axis_name="s")
    out_type = (
        jax.ShapeDtypeStruct((B, D), F32),
        jax.ShapeDtypeStruct((B, D), F32),
        jax.ShapeDtypeStruct((BL, D), F32),
        jax.ShapeDtypeStruct((BL, D), F32),
        jax.ShapeDtypeStruct((BL, D), F32),
        jax.ShapeDtypeStruct((BL, D), F32),
    )

    @functools.partial(
        pl.kernel,
        mesh=mesh,
        out_type=out_type,
        scratch_types=[
            pltpu.VMEM((CH,), jnp.int32),
            pltpu.VMEM((CH, D), F32),
            pltpu.VMEM((CH, D), F32),
            pltpu.SemaphoreType.DMA,
            pltpu.SemaphoreType.DMA,
        ],
        compiler_params=pltpu.CompilerParams(use_tc_tiling_on_sc=False),
    )
    def k(eum, eim, eumf, eimf, uid_h, pos_h, neg_h,
          o_ue_mlp, o_ue_mf, o_pos_mlp, o_neg_mlp, o_pos_mf, o_neg_mf,
          idx_v, r0, r1, sem0, sem1):
        wid = lax.axis_index("s") * NC + lax.axis_index("c")
        jobs = (
            (uid_h, B // NW, eum, o_ue_mlp, eumf, o_ue_mf),
            (pos_h, BL // NW, eim, o_pos_mlp, eimf, o_pos_mf),
            (neg_h, BL // NW, eim, o_neg_mlp, eimf, o_neg_mf),
        )
        for idx_h, n_per_w, t0, o0, t1, o1 in jobs:
            nchunks = n_per_w // CH

            def body(c, _, idx_h=idx_h, n_per_w=n_per_w,
                     t0=t0, o0=o0, t1=t1, o1=o1):
                base = wid * n_per_w + c * CH
                pltpu.sync_copy(idx_h.at[pl.ds(base, CH)], idx_v)
                cp0 = pltpu.async_copy(t0.at[idx_v], r0, sem0)
                cp1 = pltpu.async_copy(t1.at[idx_v], r1, sem1)
                cp0.wait()
                cp1.wait()
                pltpu.sync_copy(r0, o0.at[pl.ds(base, CH)])
                pltpu.sync_copy(r1, o1.at[pl.ds(base, CH)])
                return 0

            lax.fori_loop(0, nchunks, body, 0)

    return k(Eum, Eim, Eumf, Eimf, uid, posf, negf)


BB = 256
RR = 5 * BB


def _tc_body(ue_mlp, ue_mf, pos_mlp, neg_mlp, pos_mf, neg_mf,
             w0ut, a0, a1, a2, aom, aof, b0t, b1t, b2t, bot,
             pos_out, neg_out):
    hu = jnp.dot(ue_mlp[:], w0ut[:], preferred_element_type=F32)
    hu4 = jnp.concatenate([hu, hu, hu, hu], axis=1)
    hu_rep = jnp.broadcast_to(hu4[:, None, :], (BB, 5, 4 * D)).reshape(RR, 4 * D)
    uf = ue_mf[:]
    uf4 = jnp.concatenate([uf, uf, uf, uf], axis=1)
    uf_rep = jnp.broadcast_to(uf4[:, None, :], (BB, 5, 4 * D)).reshape(RR, 4 * D)

    def tower(ie_mlp, ie_mf):
        h = jnp.maximum(
            jnp.dot(ie_mlp, a0[:], preferred_element_type=F32) + hu_rep + b0t[:], 0.0)
        h = jnp.maximum(
            jnp.dot(h, a1[:], preferred_element_type=F32) + b1t[:], 0.0)
        h = jnp.maximum(
            jnp.dot(h, a2[:], preferred_element_type=F32) + b2t[:], 0.0)
        return (jnp.dot(h, aom[:], preferred_element_type=F32)
                + jnp.dot(uf_rep * ie_mf, aof[:], preferred_element_type=F32)
                + bot[:])

    pos_out[:] = tower(pos_mlp[:], pos_mf[:])
    neg_out[:] = tower(neg_mlp[:], neg_mf[:])


def kernel(Eum, Eim, Eumf, Eimf, W0, b0, W1, b1, W2, b2, Wo, bo, uid, pos, neg):
    posf = pos.reshape(-1)
    negf = neg.reshape(-1)
    ue_mlp, ue_mf, pos_mlp, neg_mlp, pos_mf, neg_mf = _sc_gather(
        Eum, Eim, Eumf, Eimf, uid, posf, negf)

    pos_mlp_p = pos_mlp.reshape(BL // 4, 4 * D)
    neg_mlp_p = neg_mlp.reshape(BL // 4, 4 * D)
    pos_mf_p = pos_mf.reshape(BL // 4, 4 * D)
    neg_mf_p = neg_mf.reshape(BL // 4, 4 * D)

    eye4 = jnp.eye(4, dtype=F32)
    w0ut = W0[:, :D].T
    a0 = jnp.kron(eye4, W0[:, D:].T)
    a1 = jnp.kron(eye4, W1.T)
    a2 = jnp.kron(eye4, W2.T)
    aom = jnp.kron(eye4, Wo[:, :D].T)
    aof = jnp.kron(eye4, Wo[:, D:].T)
    b0t = jnp.tile(b0, 4)[None]
    b1t = jnp.tile(b1, 4)[None]
    b2t = jnp.tile(b2, 4)[None]
    bot = jnp.tile(bo, 4)[None]

    grid = B // BB
    full = lambda shape: pl.BlockSpec(shape, lambda i: (0, 0))
    pos_out, neg_out = pl.pallas_call(
        _tc_body,
        grid=(grid,),
        in_specs=[
            pl.BlockSpec((BB, D), lambda i: (i, 0)),
            pl.BlockSpec((BB, D), lambda i: (i, 0)),
            pl.BlockSpec((RR, 4 * D), lambda i: (i, 0)),
            pl.BlockSpec((RR, 4 * D), lambda i: (i, 0)),
            pl.BlockSpec((RR, 4 * D), lambda i: (i, 0)),
            pl.BlockSpec((RR, 4 * D), lambda i: (i, 0)),
            full((D, D)), full((4 * D, 4 * D)), full((4 * D, 4 * D)),
            full((4 * D, 4 * D)), full((4 * D, 4)), full((4 * D, 4)),
            full((1, 4 * D)), full((1, 4 * D)), full((1, 4 * D)), full((1, 4)),
        ],
        out_specs=[
            pl.BlockSpec((RR, 4), lambda i: (i, 0)),
            pl.BlockSpec((RR, 4), lambda i: (i, 0)),
        ],
        out_shape=[
            jax.ShapeDtypeStruct((BL // 4, 4), F32),
            jax.ShapeDtypeStruct((BL // 4, 4), F32),
        ],
        compiler_params=pltpu.CompilerParams(
            dimension_semantics=("arbitrary",),
        ),
    )(ue_mlp, ue_mf, pos_mlp_p, neg_mlp_p, pos_mf_p, neg_mf_p,
      w0ut, a0, a1, a2, aom, aof, b0t, b1t, b2t, bot)

    return (pos_out.reshape(B, L, 1), neg_out.reshape(B, L, 1))


# speedup vs baseline: 17.1191x; 1.0822x over previous
import functools

import jax
import jax.numpy as jnp
from jax import lax
from jax.experimental import pallas as pl
from jax.experimental.pallas import tpu as pltpu
from jax.experimental.pallas import tpu_sc as plsc

B = 16384
L = 20
V = 1000000
D = 32
BL = B * L

NC = 2
NS = 16
NW = NC * NS

CH = 512

F32 = jnp.float32


def _sc_gather2(tz_hbm, idx, n_out):
    mesh = plsc.VectorSubcoreMesh(core_axis_name="c", subcore_axis_name="s")
    out_type = (
        jax.ShapeDtypeStruct((n_out, D), F32),
        jax.ShapeDtypeStruct((n_out, D), F32),
    )

    @functools.partial(
        pl.kernel,
        mesh=mesh,
        out_type=out_type,
        scratch_types=[
            pltpu.VMEM((CH,), jnp.int32),
            pltpu.VMEM((CH, 2 * D), F32),
            pltpu.SemaphoreType.DMA,
        ],
        compiler_params=pltpu.CompilerParams(use_tc_tiling_on_sc=False),
    )
    def k(tz, idx_h, o0, o1, idx_v, rz, sem):
        wid = lax.axis_index("s") * NC + lax.axis_index("c")
        n_per_w = n_out // NW
        nchunks = n_per_w // CH

        def body(c, _):
            base = wid * n_per_w + c * CH
            pltpu.sync_copy(idx_h.at[pl.ds(base, CH)], idx_v)
            pltpu.async_copy(tz.at[idx_v], rz, sem).wait()
            pltpu.sync_copy(rz.at[:, pl.ds(0, D)], o0.at[pl.ds(base, CH)])
            pltpu.sync_copy(rz.at[:, pl.ds(D, D)], o1.at[pl.ds(base, CH)])
            return 0

        lax.fori_loop(0, nchunks, body, 0)

    return k(tz_hbm, idx)


TB = 8192


def _tp_body(xa_ref, xb_ref, eye_ref, o_ref):
    x = jnp.concatenate([xa_ref[:], xb_ref[:]], axis=0)
    o_ref[:] = lax.dot_general(
        x, eye_ref[:], (((0,), (0,)), ((), ())),
        preferred_element_type=F32)


def _tc_stack_transpose(ta, tb):
    eye = jnp.eye(2 * D, dtype=F32)
    return pl.pallas_call(
        _tp_body,
        grid=((V + TB - 1) // TB,),
        in_specs=[
            pl.BlockSpec((D, TB), lambda i: (0, i)),
            pl.BlockSpec((D, TB), lambda i: (0, i)),
            pl.BlockSpec((2 * D, 2 * D), lambda i: (0, 0)),
        ],
        out_specs=pl.BlockSpec((TB, 2 * D), lambda i: (i, 0)),
        out_shape=jax.ShapeDtypeStruct((V, 2 * D), F32),
        compiler_params=pltpu.CompilerParams(
            dimension_semantics=("arbitrary",),
        ),
    )(ta.T, tb.T, eye)


BB = 256
RR = 5 * BB


def _tc_body(ue_mlp, ue_mf, pos_mlp, neg_mlp, pos_mf, neg_mf,
             w0ut, a0, a1, a2, aom, aof, b0r, b1t, b2t, bot,
             pos_out, neg_out):
    hu = jnp.dot(ue_mlp[:], w0ut[:], preferred_element_type=F32) + b0r[:]
    hu4 = jnp.concatenate([hu, hu, hu, hu], axis=1)
    hu_rep = jnp.broadcast_to(hu4[:, None, :], (BB, 5, 4 * D)).reshape(RR, 4 * D)
    uf = ue_mf[:]
    uf4 = jnp.concatenate([uf, uf, uf, uf], axis=1)
    uf_rep = jnp.broadcast_to(uf4[:, None, :], (BB, 5, 4 * D)).reshape(RR, 4 * D)

    def tower(ie_mlp, ie_mf):
        h = jnp.maximum(
            jnp.dot(ie_mlp, a0[:], preferred_element_type=F32) + hu_rep, 0.0)
        h = jnp.maximum(
            jnp.dot(h, a1[:], preferred_element_type=F32) + b1t[:], 0.0)
        h = jnp.maximum(
            jnp.dot(h, a2[:], preferred_element_type=F32) + b2t[:], 0.0)
        return (jnp.dot(h, aom[:], preferred_element_type=F32)
                + jnp.dot(uf_rep * ie_mf, aof[:], preferred_element_type=F32)
                + bot[:])

    pos_out[:] = tower(pos_mlp[:], pos_mf[:])
    neg_out[:] = tower(neg_mlp[:], neg_mf[:])


def kernel(Eum, Eim, Eumf, Eimf, W0, b0, W1, b1, W2, b2, Wo, bo, uid, pos, neg):
    posf = pos.reshape(-1)
    negf = neg.reshape(-1)

    Z_item = _tc_stack_transpose(Eim, Eimf)
    Z_user = _tc_stack_transpose(Eum, Eumf)

    pos_mlp, pos_mf = _sc_gather2(Z_item, posf, BL)
    neg_mlp, neg_mf = _sc_gather2(Z_item, negf, BL)
    ue_mlp, ue_mf = _sc_gather2(Z_user, uid, B)

    pos_mlp_p = pos_mlp.reshape(BL // 4, 4 * D)
    neg_mlp_p = neg_mlp.reshape(BL // 4, 4 * D)
    pos_mf_p = pos_mf.reshape(BL // 4, 4 * D)
    neg_mf_p = neg_mf.reshape(BL // 4, 4 * D)

    eye4 = jnp.eye(4, dtype=F32)
    w0ut = W0[:, :D].T
    a0 = jnp.kron(eye4, W0[:, D:].T)
    a1 = jnp.kron(eye4, W1.T)
    a2 = jnp.kron(eye4, W2.T)
    aom = jnp.kron(eye4, Wo[:, :D].T)
    aof = jnp.kron(eye4, Wo[:, D:].T)
    b0r = b0[None]
    b1t = jnp.tile(b1, 4)[None]
    b2t = jnp.tile(b2, 4)[None]
    bot = jnp.tile(bo, 4)[None]

    grid = B // BB
    full = lambda shape: pl.BlockSpec(shape, lambda i: (0, 0))
    pos_out, neg_out = pl.pallas_call(
        _tc_body,
        grid=(grid,),
        in_specs=[
            pl.BlockSpec((BB, D), lambda i: (i, 0)),
            pl.BlockSpec((BB, D), lambda i: (i, 0)),
            pl.BlockSpec((RR, 4 * D), lambda i: (i, 0)),
            pl.BlockSpec((RR, 4 * D), lambda i: (i, 0)),
            pl.BlockSpec((RR, 4 * D), lambda i: (i, 0)),
            pl.BlockSpec((RR, 4 * D), lambda i: (i, 0)),
            full((D, D)), full((4 * D, 4 * D)), full((4 * D, 4 * D)),
            full((4 * D, 4 * D)), full((4 * D, 4)), full((4 * D, 4)),
            full((1, D)), full((1, 4 * D)), full((1, 4 * D)), full((1, 4)),
        ],
        out_specs=[
            pl.BlockSpec((RR, 4), lambda i: (i, 0)),
            pl.BlockSpec((RR, 4), lambda i: (i, 0)),
        ],
        out_shape=[
            jax.ShapeDtypeStruct((BL // 4, 4), F32),
            jax.ShapeDtypeStruct((BL // 4, 4), F32),
        ],
        compiler_params=pltpu.CompilerParams(
            dimension_semantics=("arbitrary",),
        ),
    )(ue_mlp, ue_mf, pos_mlp_p, neg_mlp_p, pos_mf_p, neg_mf_p,
      w0ut, a0, a1, a2, aom, aof, b0r, b1t, b2t, bot)

    return (pos_out.reshape(B, L, 1), neg_out.reshape(B, L, 1))


# speedup vs baseline: 17.1570x; 1.0022x over previous
import functools

import numpy as np

import jax
import jax.numpy as jnp
from jax import lax
from jax.experimental import pallas as pl
from jax.experimental.pallas import tpu as pltpu
from jax.experimental.pallas import tpu_sc as plsc

B = 16384
L = 20
V = 1000000
D = 32
BL = B * L

NC = 2
NS = 16
NW = NC * NS

CH = 512

F32 = jnp.float32


DW = D // 2


def _sc_gather2(tz_hbm, idx, n_out):
    mesh = plsc.VectorSubcoreMesh(core_axis_name="c", subcore_axis_name="s")
    out_type = (
        jax.ShapeDtypeStruct((n_out, DW), jnp.int32),
        jax.ShapeDtypeStruct((n_out, DW), jnp.int32),
    )

    @functools.partial(
        pl.kernel,
        mesh=mesh,
        out_type=out_type,
        scratch_types=[
            pltpu.VMEM((CH,), jnp.int32),
            pltpu.VMEM((CH, D), jnp.int32),
            pltpu.SemaphoreType.DMA,
        ],
        compiler_params=pltpu.CompilerParams(use_tc_tiling_on_sc=False),
    )
    def k(tz, idx_h, o0, o1, idx_v, rz, sem):
        wid = lax.axis_index("s") * NC + lax.axis_index("c")
        n_per_w = n_out // NW
        nchunks = n_per_w // CH

        def body(c, _):
            base = wid * n_per_w + c * CH
            pltpu.sync_copy(idx_h.at[pl.ds(base, CH)], idx_v)
            pltpu.async_copy(tz.at[idx_v], rz, sem).wait()
            pltpu.sync_copy(rz.at[:, pl.ds(0, DW)], o0.at[pl.ds(base, CH)])
            pltpu.sync_copy(rz.at[:, pl.ds(DW, DW)], o1.at[pl.ds(base, CH)])
            return 0

        lax.fori_loop(0, nchunks, body, 0)

    return k(tz_hbm, idx)


TB = 8192


_HI = np.uint32(0xFFFF0000)


def _tp_body(xa_ref, xb_ref, se_ref, so_ref, o_ref):
    x = jnp.concatenate([xa_ref[:], xb_ref[:]], axis=0)
    ye = lax.dot_general(
        x, se_ref[:], (((0,), (0,)), ((), ())), preferred_element_type=F32)
    yo = lax.dot_general(
        x, so_ref[:], (((0,), (0,)), ((), ())), preferred_element_type=F32)
    re = lax.bitcast_convert_type(
        ye.astype(jnp.bfloat16).astype(F32), jnp.uint32)
    ro = lax.bitcast_convert_type(
        yo.astype(jnp.bfloat16).astype(F32), jnp.uint32)
    w = (re >> 16) | (ro & _HI)
    o_ref[:] = lax.bitcast_convert_type(w, jnp.int32)


def _tc_stack_transpose(ta, tb):
    eye = jnp.eye(2 * D, dtype=F32)
    se = eye[:, 0::2]
    so = eye[:, 1::2]
    return pl.pallas_call(
        _tp_body,
        grid=((V + TB - 1) // TB,),
        in_specs=[
            pl.BlockSpec((D, TB), lambda i: (0, i)),
            pl.BlockSpec((D, TB), lambda i: (0, i)),
            pl.BlockSpec((2 * D, D), lambda i: (0, 0)),
            pl.BlockSpec((2 * D, D), lambda i: (0, 0)),
        ],
        out_specs=pl.BlockSpec((TB, D), lambda i: (i, 0)),
        out_shape=jax.ShapeDtypeStruct((V, D), jnp.int32),
        compiler_params=pltpu.CompilerParams(
            dimension_semantics=("arbitrary",),
        ),
    )(ta.T, tb.T, se, so)


BB = 256
RR = 5 * BB


def _unpack(xi):
    u = lax.bitcast_convert_type(xi, jnp.uint32)
    lo = lax.bitcast_convert_type(u << 16, F32)
    hi = lax.bitcast_convert_type(u & _HI, F32)
    return jnp.concatenate([lo, hi], axis=1)


def _tc_body(ue_mlp, ue_mf, pos_mlp, neg_mlp, pos_mf, neg_mf,
             w0ut, a0, a1, a2, aom, aof, b0r, b1t, b2t, bot,
             pos_out, neg_out):
    hu = jnp.dot(_unpack(ue_mlp[:]), w0ut[:],
                 preferred_element_type=F32) + b0r[:]
    hu4 = jnp.concatenate([hu, hu, hu, hu], axis=1)
    hu_rep = jnp.broadcast_to(hu4[:, None, :], (BB, 5, 4 * D)).reshape(RR, 4 * D)
    uf = _unpack(ue_mf[:])
    ulo, uhi = uf[:, :DW], uf[:, DW:]
    uf4 = jnp.concatenate([ulo, ulo, ulo, ulo, uhi, uhi, uhi, uhi], axis=1)
    uf_rep = jnp.broadcast_to(uf4[:, None, :], (BB, 5, 4 * D)).reshape(RR, 4 * D)

    def tower(ie_mlp, ie_mf):
        h = jnp.maximum(
            jnp.dot(ie_mlp, a0[:], preferred_element_type=F32) + hu_rep, 0.0)
        h = jnp.maximum(
            jnp.dot(h, a1[:], preferred_element_type=F32) + b1t[:], 0.0)
        h = jnp.maximum(
            jnp.dot(h, a2[:], preferred_element_type=F32) + b2t[:], 0.0)
        return (jnp.dot(h, aom[:], preferred_element_type=F32)
                + jnp.dot(uf_rep * ie_mf, aof[:], preferred_element_type=F32)
                + bot[:])

    pos_out[:] = tower(_unpack(pos_mlp[:]), _unpack(pos_mf[:]))
    neg_out[:] = tower(_unpack(neg_mlp[:]), _unpack(neg_mf[:]))


def kernel(Eum, Eim, Eumf, Eimf, W0, b0, W1, b1, W2, b2, Wo, bo, uid, pos, neg):
    posf = pos.reshape(-1)
    negf = neg.reshape(-1)

    Z_item = _tc_stack_transpose(Eim, Eimf)
    Z_user = _tc_stack_transpose(Eum, Eumf)

    pos_mlp, pos_mf = _sc_gather2(Z_item, posf, BL)
    neg_mlp, neg_mf = _sc_gather2(Z_item, negf, BL)
    ue_mlp, ue_mf = _sc_gather2(Z_user, uid, B)

    pos_mlp_p = pos_mlp.reshape(BL // 4, 4 * DW)
    neg_mlp_p = neg_mlp.reshape(BL // 4, 4 * DW)
    pos_mf_p = pos_mf.reshape(BL // 4, 4 * DW)
    neg_mf_p = neg_mf.reshape(BL // 4, 4 * DW)

    eye4 = jnp.eye(4, dtype=F32)
    canon32 = jnp.array(
        [2 * l for l in range(DW)] + [2 * l + 1 for l in range(DW)],
        dtype=jnp.int32)
    canon128 = jnp.array(
        [32 * (l // DW) + 2 * (l % DW) for l in range(4 * DW)]
        + [32 * (l // DW) + 2 * (l % DW) + 1 for l in range(4 * DW)],
        dtype=jnp.int32)
    w0ut = W0[:, :D].T[canon32]
    a0 = jnp.kron(eye4, W0[:, D:].T)[canon128]
    a1 = jnp.kron(eye4, W1.T)
    a2 = jnp.kron(eye4, W2.T)
    aom = jnp.kron(eye4, Wo[:, :D].T)
    aof = jnp.kron(eye4, Wo[:, D:].T)[canon128]
    b0r = b0[None]
    b1t = jnp.tile(b1, 4)[None]
    b2t = jnp.tile(b2, 4)[None]
    bot = jnp.tile(bo, 4)[None]

    grid = B // BB
    full = lambda shape: pl.BlockSpec(shape, lambda i: (0, 0))
    pos_out, neg_out = pl.pallas_call(
        _tc_body,
        grid=(grid,),
        in_specs=[
            pl.BlockSpec((BB, DW), lambda i: (i, 0)),
            pl.BlockSpec((BB, DW), lambda i: (i, 0)),
            pl.BlockSpec((RR, 4 * DW), lambda i: (i, 0)),
            pl.BlockSpec((RR, 4 * DW), lambda i: (i, 0)),
            pl.BlockSpec((RR, 4 * DW), lambda i: (i, 0)),
            pl.BlockSpec((RR, 4 * DW), lambda i: (i, 0)),
            full((D, D)), full((4 * D, 4 * D)), full((4 * D, 4 * D)),
            full((4 * D, 4 * D)), full((4 * D, 4)), full((4 * D, 4)),
            full((1, D)), full((1, 4 * D)), full((1, 4 * D)), full((1, 4)),
        ],
        out_specs=[
            pl.BlockSpec((RR, 4), lambda i: (i, 0)),
            pl.BlockSpec((RR, 4), lambda i: (i, 0)),
        ],
        out_shape=[
            jax.ShapeDtypeStruct((BL // 4, 4), F32),
            jax.ShapeDtypeStruct((BL // 4, 4), F32),
        ],
        compiler_params=pltpu.CompilerParams(
            dimension_semantics=("arbitrary",),
        ),
    )(ue_mlp, ue_mf, pos_mlp_p, neg_mlp_p, pos_mf_p, neg_mf_p,
      w0ut, a0, a1, a2, aom, aof, b0r, b1t, b2t, bot)

    return (pos_out.reshape(B, L, 1), neg_out.reshape(B, L, 1))


# speedup vs baseline: 17.5747x; 1.0243x over previous
import functools

import numpy as np

import jax
import jax.numpy as jnp
from jax import lax
from jax.experimental import pallas as pl
from jax.experimental.pallas import tpu as pltpu
from jax.experimental.pallas import tpu_sc as plsc

B = 16384
L = 20
V = 1000000
D = 32
BL = B * L

NC = 2
NS = 16
NW = NC * NS

CH = 512

F32 = jnp.float32
_HI = np.uint32(0xFFFF0000)


def _sc_gather_fused(tz_hbm, idx, n_out):
    mesh = plsc.VectorSubcoreMesh(core_axis_name="c", subcore_axis_name="s")

    @functools.partial(
        pl.kernel,
        mesh=mesh,
        out_type=jax.ShapeDtypeStruct((n_out, D), jnp.int32),
        scratch_types=[
            pltpu.VMEM((CH,), jnp.int32),
            pltpu.VMEM((CH, D), jnp.int32),
            pltpu.SemaphoreType.DMA,
        ],
        compiler_params=pltpu.CompilerParams(use_tc_tiling_on_sc=False),
    )
    def k(tz, idx_h, o, idx_v, rz, sem):
        wid = lax.axis_index("s") * NC + lax.axis_index("c")
        n_per_w = n_out // NW
        nchunks = n_per_w // CH

        def body(c, _):
            base = wid * n_per_w + c * CH
            pltpu.sync_copy(idx_h.at[pl.ds(base, CH)], idx_v)
            pltpu.async_copy(tz.at[idx_v], rz, sem).wait()
            pltpu.sync_copy(rz, o.at[pl.ds(base, CH)])
            return 0

        lax.fori_loop(0, nchunks, body, 0)

    return k(tz_hbm, idx)


def _sc_gather_user(t0_hbm, t1_hbm, uid):
    mesh = plsc.VectorSubcoreMesh(core_axis_name="c", subcore_axis_name="s")
    out_type = (
        jax.ShapeDtypeStruct((B, D), F32),
        jax.ShapeDtypeStruct((B, D), F32),
    )

    @functools.partial(
        pl.kernel,
        mesh=mesh,
        out_type=out_type,
        scratch_types=[
            pltpu.VMEM((CH,), jnp.int32),
            pltpu.VMEM((CH, D), F32),
            pltpu.VMEM((CH, D), F32),
            pltpu.SemaphoreType.DMA,
            pltpu.SemaphoreType.DMA,
        ],
        compiler_params=pltpu.CompilerParams(use_tc_tiling_on_sc=False),
    )
    def k(t0, t1, idx_h, o0, o1, idx_v, r0, r1, sem0, sem1):
        wid = lax.axis_index("s") * NC + lax.axis_index("c")
        n_per_w = B // NW
        nchunks = n_per_w // CH

        def body(c, _):
            base = wid * n_per_w + c * CH
            pltpu.sync_copy(idx_h.at[pl.ds(base, CH)], idx_v)
            cp0 = pltpu.async_copy(t0.at[idx_v], r0, sem0)
            cp1 = pltpu.async_copy(t1.at[idx_v], r1, sem1)
            cp0.wait()
            cp1.wait()
            pltpu.sync_copy(r0, o0.at[pl.ds(base, CH)])
            pltpu.sync_copy(r1, o1.at[pl.ds(base, CH)])
            return 0

        lax.fori_loop(0, nchunks, body, 0)

    return k(t0_hbm, t1_hbm, uid)


TB = 16384


def _tp_body(xa_ref, xb_ref, se_ref, so_ref, o_ref):
    x = jnp.concatenate([xa_ref[:], xb_ref[:]], axis=0)
    ye = lax.dot_general(
        x, se_ref[:], (((0,), (0,)), ((), ())), preferred_element_type=F32)
    yo = lax.dot_general(
        x, so_ref[:], (((0,), (0,)), ((), ())), preferred_element_type=F32)
    re = lax.bitcast_convert_type(
        ye.astype(jnp.bfloat16).astype(F32), jnp.uint32)
    ro = lax.bitcast_convert_type(
        yo.astype(jnp.bfloat16).astype(F32), jnp.uint32)
    w = (re >> 16) | (ro & _HI)
    o_ref[:] = lax.bitcast_convert_type(w, jnp.int32)


def _tc_stack_transpose(ta, tb):
    eye = jnp.eye(2 * D, dtype=F32)
    se = eye[:, 0::2]
    so = eye[:, 1::2]
    return pl.pallas_call(
        _tp_body,
        grid=((V + TB - 1) // TB,),
        in_specs=[
            pl.BlockSpec((D, TB), lambda i: (0, i)),
            pl.BlockSpec((D, TB), lambda i: (0, i)),
            pl.BlockSpec((2 * D, D), lambda i: (0, 0)),
            pl.BlockSpec((2 * D, D), lambda i: (0, 0)),
        ],
        out_specs=pl.BlockSpec((TB, D), lambda i: (i, 0)),
        out_shape=jax.ShapeDtypeStruct((V, D), jnp.int32),
        compiler_params=pltpu.CompilerParams(
            dimension_semantics=("arbitrary",),
        ),
    )(ta.T, tb.T, se, so)


BB = 256
RR = 5 * BB


def _unpack(xi):
    u = lax.bitcast_convert_type(xi, jnp.uint32)
    lo = lax.bitcast_convert_type(u << 16, F32)
    hi = lax.bitcast_convert_type(u & _HI, F32)
    return jnp.concatenate([lo, hi], axis=1)


def _tc_body(ue_mlp, ue_mf, pos_cat, neg_cat,
             w0ut, a0x, a1, a2, aom, aofx, su, b0r, b1t, b2t, botc,
             pos_out, neg_out):
    hu = jnp.dot(ue_mlp[:], w0ut[:], preferred_element_type=F32) + b0r[:]
    hu4 = jnp.concatenate([hu, hu, hu, hu], axis=1)
    hu_rep = jnp.broadcast_to(hu4[:, None, :], (BB, 5, 4 * D)).reshape(RR, 4 * D)
    ufx = jnp.dot(ue_mf[:], su[:], preferred_element_type=F32)
    uf_rep = jnp.broadcast_to(ufx[:, None, :], (BB, 5, 8 * D)).reshape(RR, 8 * D)

    def tower(vcat):
        v = _unpack(vcat)
        h = jnp.maximum(
            jnp.dot(v, a0x[:], preferred_element_type=F32) + hu_rep, 0.0)
        h = jnp.maximum(
            jnp.dot(h, a1[:], preferred_element_type=F32) + b1t[:], 0.0)
        h = jnp.maximum(
            jnp.dot(h, a2[:], preferred_element_type=F32) + b2t[:], 0.0)
        lg = lax.dot_general(
            aom[:], h, (((0,), (1,)), ((), ())),
            preferred_element_type=F32)
        mf = lax.dot_general(
            aofx[:], v * uf_rep, (((0,), (1,)), ((), ())),
            preferred_element_type=F32)
        return lg + mf + botc[:]

    pos_out[:] = tower(pos_cat[:])
    neg_out[:] = tower(neg_cat[:])


def kernel(Eum, Eim, Eumf, Eimf, W0, b0, W1, b1, W2, b2, Wo, bo, uid, pos, neg):
    posf = pos.reshape(-1)
    negf = neg.reshape(-1)

    Z_item = _tc_stack_transpose(Eim, Eimf)

    pos_cat = _sc_gather_fused(Z_item, posf, BL)
    neg_cat = _sc_gather_fused(Z_item, negf, BL)
    ue_mlp, ue_mf = _sc_gather_user(Eum, Eumf, uid)

    pos_p = pos_cat.reshape(BL // 4, 4 * D)
    neg_p = neg_cat.reshape(BL // 4, 4 * D)

    eye4 = jnp.eye(4, dtype=F32)
    fperm = np.array(
        [64 * ((l % 128) // 32) + 2 * (l % 32) + (l >= 128)
         for l in range(256)])
    a0_base = jnp.kron(eye4, jnp.concatenate(
        [W0[:, D:].T, jnp.zeros((D, D), F32)], axis=0))
    a0x = a0_base[fperm]
    aof_base = jnp.kron(eye4, jnp.concatenate(
        [jnp.zeros((D, 1), F32), Wo[:, D:].T], axis=0))
    aofx = aof_base[fperm]
    su_np = np.zeros((D, 256), np.float32)
    for l_ in range(256):
        f_ = 64 * 0 + 2 * (l_ % 32) + (1 if l_ >= 128 else 0)
        if f_ >= 32:
            su_np[f_ - 32, l_] = 1.0
    su = jnp.asarray(su_np)
    w0ut = W0[:, :D].T
    a1 = jnp.kron(eye4, W1.T)
    a2 = jnp.kron(eye4, W2.T)
    aom = jnp.kron(eye4, Wo[:, :D].T)
    b0r = b0[None]
    b1t = jnp.tile(b1, 4)[None]
    b2t = jnp.tile(b2, 4)[None]
    botc = jnp.tile(bo, 4)[:, None]

    grid = B // BB
    full = lambda shape: pl.BlockSpec(shape, lambda i: (0, 0))
    pos_t, neg_t = pl.pallas_call(
        _tc_body,
        grid=(grid,),
        in_specs=[
            pl.BlockSpec((BB, D), lambda i: (i, 0)),
            pl.BlockSpec((BB, D), lambda i: (i, 0)),
            pl.BlockSpec((RR, 4 * D), lambda i: (i, 0)),
            pl.BlockSpec((RR, 4 * D), lambda i: (i, 0)),
            full((D, D)), full((8 * D, 4 * D)), full((4 * D, 4 * D)),
            full((4 * D, 4 * D)), full((4 * D, 4)), full((8 * D, 4)),
            full((D, 8 * D)),
            full((1, D)), full((1, 4 * D)), full((1, 4 * D)), full((4, 1)),
        ],
        out_specs=[
            pl.BlockSpec((4, RR), lambda i: (0, i)),
            pl.BlockSpec((4, RR), lambda i: (0, i)),
        ],
        out_shape=[
            jax.ShapeDtypeStruct((4, BL // 4), F32),
            jax.ShapeDtypeStruct((4, BL // 4), F32),
        ],
        compiler_params=pltpu.CompilerParams(
            dimension_semantics=("arbitrary",),
        ),
    )(ue_mlp, ue_mf, pos_p, neg_p,
      w0ut, a0x, a1, a2, aom, aofx, su, b0r, b1t, b2t, botc)

    return (pos_t.T.reshape(B, L, 1), neg_t.T.reshape(B, L, 1))


# speedup vs baseline: 20.5971x; 1.1720x over previous
import functools

import numpy as np

import jax
import jax.numpy as jnp
from jax import lax
from jax.experimental import pallas as pl
from jax.experimental.pallas import tpu as pltpu
from jax.experimental.pallas import tpu_sc as plsc

B = 16384
L = 20
V = 1000000
D = 32
BL = B * L

NC = 2
NS = 16
NW = NC * NS

CH = 512

F32 = jnp.float32
_HI = np.uint32(0xFFFF0000)


def _sc_gather_all(tz_item, tz_user, posf, negf, uid):
    mesh = plsc.VectorSubcoreMesh(core_axis_name="c", subcore_axis_name="s")
    out_type = (
        jax.ShapeDtypeStruct((BL, D), jnp.int32),
        jax.ShapeDtypeStruct((BL, D), jnp.int32),
        jax.ShapeDtypeStruct((B, D), jnp.int32),
    )

    @functools.partial(
        pl.kernel,
        mesh=mesh,
        out_type=out_type,
        scratch_types=[
            pltpu.VMEM((CH,), jnp.int32),
            pltpu.VMEM((CH, D), jnp.int32),
            pltpu.SemaphoreType.DMA,
        ],
        compiler_params=pltpu.CompilerParams(use_tc_tiling_on_sc=False),
    )
    def k(tzi, tzu, pos_h, neg_h, uid_h, o_pos, o_neg, o_ue, idx_v, rz, sem):
        wid = lax.axis_index("s") * NC + lax.axis_index("c")
        jobs = (
            (pos_h, BL // NW, tzi, o_pos),
            (neg_h, BL // NW, tzi, o_neg),
            (uid_h, B // NW, tzu, o_ue),
        )
        for idx_h, n_per_w, tz, o in jobs:
            nchunks = n_per_w // CH

            def body(c, _, idx_h=idx_h, n_per_w=n_per_w, tz=tz, o=o):
                base = wid * n_per_w + c * CH
                pltpu.sync_copy(idx_h.at[pl.ds(base, CH)], idx_v)
                pltpu.async_copy(tz.at[idx_v], rz, sem).wait()
                pltpu.sync_copy(rz, o.at[pl.ds(base, CH)])
                return 0

            lax.fori_loop(0, nchunks, body, 0)

    return k(tz_item, tz_user, posf, negf, uid)


TB = 16384


def _tp_body(xa_ref, xb_ref, se_ref, so_ref, o_ref):
    x = jnp.concatenate([xa_ref[:], xb_ref[:]], axis=0)
    ye = lax.dot_general(
        x, se_ref[:], (((0,), (0,)), ((), ())), preferred_element_type=F32)
    yo = lax.dot_general(
        x, so_ref[:], (((0,), (0,)), ((), ())), preferred_element_type=F32)
    re = lax.bitcast_convert_type(
        ye.astype(jnp.bfloat16).astype(F32), jnp.uint32)
    ro = lax.bitcast_convert_type(
        yo.astype(jnp.bfloat16).astype(F32), jnp.uint32)
    w = (re >> 16) | (ro & _HI)
    o_ref[:] = lax.bitcast_convert_type(w, jnp.int32)


def _tc_stack_transpose(ta, tb):
    eye = jnp.eye(2 * D, dtype=F32)
    se = eye[:, 0::2]
    so = eye[:, 1::2]
    return pl.pallas_call(
        _tp_body,
        grid=((V + TB - 1) // TB,),
        in_specs=[
            pl.BlockSpec((D, TB), lambda i: (0, i)),
            pl.BlockSpec((D, TB), lambda i: (0, i)),
            pl.BlockSpec((2 * D, D), lambda i: (0, 0)),
            pl.BlockSpec((2 * D, D), lambda i: (0, 0)),
        ],
        out_specs=pl.BlockSpec((TB, D), lambda i: (i, 0)),
        out_shape=jax.ShapeDtypeStruct((V, D), jnp.int32),
        compiler_params=pltpu.CompilerParams(
            dimension_semantics=("arbitrary",),
        ),
    )(ta.T, tb.T, se, so)


BB = 256
RR = 5 * BB


def _unpack(xi):
    u = lax.bitcast_convert_type(xi, jnp.uint32)
    lo = lax.bitcast_convert_type(u << 16, F32)
    hi = lax.bitcast_convert_type(u & _HI, F32)
    return jnp.concatenate([lo, hi], axis=1)


def _tc_body(ue_cat, pos_cat, neg_cat,
             w0uf, a0x, a1, a2, aom, aofx, su2, b0r, b1t, b2t, botc,
             pos_out, neg_out):
    u = _unpack(ue_cat[:])
    hu = jnp.dot(u, w0uf[:], preferred_element_type=F32) + b0r[:]
    hu4 = jnp.concatenate([hu, hu, hu, hu], axis=1)
    hu_rep = jnp.broadcast_to(hu4[:, None, :], (BB, 5, 4 * D)).reshape(RR, 4 * D)
    ufx = jnp.dot(u, su2[:], preferred_element_type=F32)
    uf_rep = jnp.broadcast_to(ufx[:, None, :], (BB, 5, 8 * D)).reshape(RR, 8 * D)

    def tower(vcat):
        v = _unpack(vcat)
        h = jnp.maximum(
            jnp.dot(v, a0x[:], preferred_element_type=F32) + hu_rep, 0.0)
        h = jnp.maximum(
            jnp.dot(h, a1[:], preferred_element_type=F32) + b1t[:], 0.0)
        h = jnp.maximum(
            jnp.dot(h, a2[:], preferred_element_type=F32) + b2t[:], 0.0)
        lg = lax.dot_general(
            aom[:], h, (((0,), (1,)), ((), ())),
            preferred_element_type=F32)
        mf = lax.dot_general(
            aofx[:], v * uf_rep, (((0,), (1,)), ((), ())),
            preferred_element_type=F32)
        return lg + mf + botc[:]

    pos_out[:] = tower(pos_cat[:])
    neg_out[:] = tower(neg_cat[:])


def kernel(Eum, Eim, Eumf, Eimf, W0, b0, W1, b1, W2, b2, Wo, bo, uid, pos, neg):
    posf = pos.reshape(-1)
    negf = neg.reshape(-1)

    Z_item = _tc_stack_transpose(Eim, Eimf)
    Z_user = _tc_stack_transpose(Eum, Eumf)

    pos_cat, neg_cat, ue_cat = _sc_gather_all(Z_item, Z_user, posf, negf, uid)

    pos_p = pos_cat.reshape(BL // 4, 4 * D)
    neg_p = neg_cat.reshape(BL // 4, 4 * D)

    eye4 = jnp.eye(4, dtype=F32)
    fperm = np.array(
        [64 * ((l % 128) // 32) + 2 * (l % 32) + (l >= 128)
         for l in range(256)])
    a0_base = jnp.kron(eye4, jnp.concatenate(
        [W0[:, D:].T, jnp.zeros((D, D), F32)], axis=0))
    a0x = a0_base[fperm]
    aof_base = jnp.kron(eye4, jnp.concatenate(
        [jnp.zeros((D, 1), F32), Wo[:, D:].T], axis=0))
    aofx = aof_base[fperm]
    ug = np.array([2 * (l_ % 32) + (1 if l_ >= 32 else 0) for l_ in range(64)])
    pu_np = np.zeros((64, D), np.float32)
    for l_ in range(64):
        if ug[l_] < D:
            pu_np[l_, ug[l_]] = 1.0
    w0uf = jnp.asarray(pu_np) @ W0[:, :D].T
    fl = np.array([2 * (l_ % 32) + (1 if l_ >= 128 else 0)
                   for l_ in range(256)])
    su2_np = np.zeros((64, 256), np.float32)
    for l_ in range(64):
        if ug[l_] >= D:
            for L_ in range(256):
                if fl[L_] == ug[l_]:
                    su2_np[l_, L_] = 1.0
    su2 = jnp.asarray(su2_np)
    a1 = jnp.kron(eye4, W1.T)
    a2 = jnp.kron(eye4, W2.T)
    aom = jnp.kron(eye4, Wo[:, :D].T)
    b0r = b0[None]
    b1t = jnp.tile(b1, 4)[None]
    b2t = jnp.tile(b2, 4)[None]
    botc = jnp.tile(bo, 4)[:, None]

    grid = B // BB
    full = lambda shape: pl.BlockSpec(shape, lambda i: (0, 0))
    pos_t, neg_t = pl.pallas_call(
        _tc_body,
        grid=(grid,),
        in_specs=[
            pl.BlockSpec((BB, D), lambda i: (i, 0)),
            pl.BlockSpec((RR, 4 * D), lambda i: (i, 0)),
            pl.BlockSpec((RR, 4 * D), lambda i: (i, 0)),
            full((2 * D, D)), full((8 * D, 4 * D)), full((4 * D, 4 * D)),
            full((4 * D, 4 * D)), full((4 * D, 4)), full((8 * D, 4)),
            full((2 * D, 8 * D)),
            full((1, D)), full((1, 4 * D)), full((1, 4 * D)), full((4, 1)),
        ],
        out_specs=[
            pl.BlockSpec((4, RR), lambda i: (0, i)),
            pl.BlockSpec((4, RR), lambda i: (0, i)),
        ],
        out_shape=[
            jax.ShapeDtypeStruct((4, BL // 4), F32),
            jax.ShapeDtypeStruct((4, BL // 4), F32),
        ],
        compiler_params=pltpu.CompilerParams(
            dimension_semantics=("arbitrary",),
        ),
    )(ue_cat, pos_p, neg_p,
      w0uf, a0x, a1, a2, aom, aofx, su2, b0r, b1t, b2t, botc)

    return (pos_t.T.reshape(B, L, 1), neg_t.T.reshape(B, L, 1))
